# baseline (speedup 1.0000x reference)
import functools
import math

import jax
import jax.numpy as jnp
from jax import lax
from jax.experimental import pallas as pl
from jax.experimental.pallas import tpu as pltpu
from jax.experimental.pallas import tpu_sc as plsc

B = 1024
V = 100000
SCALE = 64.0
COS_M2 = math.cos(0.5)
SIN_M2 = math.sin(0.5)
NEG = -1e30

C_SC = 20480

CBLK = 2048
BLK0 = C_SC // CBLK
NBLK = (V - C_SC + CBLK - 1) // CBLK

NC = 2
NS = 16
NW = NC * NS
LANES = 16
NSTRIP = B // LANES
CPW = C_SC // NW
CHC = 32
NCHK = CPW // CHC


def _tc_kernel(tgt_ref, x_ref, m_ref, s_ref, zy_ref):
    pid = pl.program_id(0)

    @pl.when(pid == 0)
    def _init():
        m_ref[:, :] = jnp.full((1, B), NEG, jnp.float32)
        s_ref[:, :] = jnp.zeros((1, B), jnp.float32)
        zy_ref[:, :] = jnp.zeros((1, B), jnp.float32)

    x = x_ref[:, :]
    rows = lax.broadcasted_iota(jnp.int32, (CBLK, B), 0)

    def _step(xm):
        m_old = m_ref[:, :]
        bm = jnp.max(xm, axis=0, keepdims=True)
        m_new = jnp.maximum(m_old, bm)
        e = jnp.exp(xm - m_new)
        s_ref[:, :] = s_ref[:, :] * jnp.exp(m_old - m_new) + jnp.sum(
            e, axis=0, keepdims=True
        )
        m_ref[:, :] = m_new

        rel = tgt_ref[:, :] - (C_SC + pid * CBLK)
        zy_ref[:, :] = zy_ref[:, :] + jnp.sum(
            jnp.where(rows == rel, x, 0.0), axis=0, keepdims=True
        )

    @pl.when(pid < NBLK - 1)
    def _full():
        _step(x)

    @pl.when(pid == NBLK - 1)
    def _tail():
        _step(jnp.where(rows < V - C_SC - pid * CBLK, x, NEG))


def _sc_kernel(ft_ref, tgt_ref, m_out, s_out, zy_out,
               chunk_a, chunk_b, tgt_v, m_acc, s_acc, zy_acc, sem_a, sem_b):
    wid = lax.axis_index("s") * NC + lax.axis_index("c")
    c0 = pl.multiple_of(wid * CPW, CPW)

    pltpu.sync_copy(tgt_ref, tgt_v)

    zero = jnp.zeros((LANES,), jnp.float32)
    neg = jnp.full((LANES,), NEG, jnp.float32)
    for i in range(NSTRIP):
        m_acc[pl.ds(i * LANES, LANES)] = neg
        s_acc[pl.ds(i * LANES, LANES)] = zero
        zy_acc[pl.ds(i * LANES, LANES)] = zero

    def _start(ch, buf, sem):
        colr = pl.multiple_of(c0 + ch * CHC, 8)
        pltpu.async_copy(ft_ref.at[pl.ds(colr, CHC), pl.ds(0, B)], buf, sem)

    def _wait(buf, sem):
        pltpu.make_async_copy(
            ft_ref.at[pl.ds(0, CHC), pl.ds(0, B)], buf, sem
        ).wait()

    def _process(chunk_v, ch):
        def strip_body(st, _):
            off = st * LANES
            m_old = m_acc[pl.ds(off, LANES)]
            s_old = s_acc[pl.ds(off, LANES)]
            z_old = zy_acc[pl.ds(off, LANES)]
            t16 = tgt_v[pl.ds(off, LANES)]
            rel16 = t16 - (c0 + ch * CHC)

            a0 = a1 = a2 = a3 = neg
            z = z_old
            for c in range(CHC):
                v = chunk_v[c, pl.ds(off, LANES)]
                z = z + jnp.where(rel16 == c, v, zero)
                if c % 4 == 0:
                    a0 = jnp.maximum(a0, v)
                elif c % 4 == 1:
                    a1 = jnp.maximum(a1, v)
                elif c % 4 == 2:
                    a2 = jnp.maximum(a2, v)
                else:
                    a3 = jnp.maximum(a3, v)
            bm = jnp.maximum(jnp.maximum(a0, a1), jnp.maximum(a2, a3))
            m_new = jnp.maximum(m_old, bm)
            s_scaled = s_old * jnp.exp(m_old - m_new)

            s0 = s1 = s2 = s3 = zero
            for c in range(CHC):
                e = jnp.exp(chunk_v[c, pl.ds(off, LANES)] - m_new)
                if c % 4 == 0:
                    s0 = s0 + e
                elif c % 4 == 1:
                    s1 = s1 + e
                elif c % 4 == 2:
                    s2 = s2 + e
                else:
                    s3 = s3 + e
            s_new = s_scaled + ((s0 + s1) + (s2 + s3))

            m_acc[pl.ds(off, LANES)] = m_new
            s_acc[pl.ds(off, LANES)] = s_new
            zy_acc[pl.ds(off, LANES)] = z
            return 0

        lax.fori_loop(0, NSTRIP, strip_body, 0)

    _start(0, chunk_a, sem_a)

    def pair_body(i, _):
        _start(2 * i + 1, chunk_b, sem_b)
        _wait(chunk_a, sem_a)
        _process(chunk_a, 2 * i)

        @pl.when(i < NCHK // 2 - 1)
        def _():
            _start(2 * i + 2, chunk_a, sem_a)

        _wait(chunk_b, sem_b)
        _process(chunk_b, 2 * i + 1)
        return 0

    lax.fori_loop(0, NCHK // 2, pair_body, 0)

    pltpu.sync_copy(m_acc, m_out.at[pl.ds(wid * B, B)])
    pltpu.sync_copy(s_acc, s_out.at[pl.ds(wid * B, B)])
    pltpu.sync_copy(zy_acc, zy_out.at[pl.ds(wid * B, B)])


def _combine_kernel(mtc_ref, stc_ref, zytc_ref, msc_ref, ssc_ref, zysc_ref,
                    out_ref):
    m16 = msc_ref[:, :]
    s16 = ssc_ref[:, :]
    m_sc = jnp.max(m16, axis=0, keepdims=True)
    s_sc = jnp.sum(s16 * jnp.exp(m16 - m_sc), axis=0, keepdims=True)
    zy = zytc_ref[:, :] + jnp.sum(zysc_ref[:, :], axis=0, keepdims=True)
    m_tc = mtc_ref[:, :]
    s_tc = stc_ref[:, :]
    m = jnp.maximum(m_tc, m_sc)
    s = s_tc * jnp.exp(m_tc - m) + s_sc * jnp.exp(m_sc - m)
    c = zy * (1.0 / SCALE)
    new_zy = SCALE * (c * COS_M2 - jnp.sqrt(1.0 - c * c) * SIN_M2)
    m2 = jnp.maximum(m, new_zy)
    inner = s * jnp.exp(m - m2) - jnp.exp(zy - m2) + jnp.exp(new_zy - m2)
    lse = m2 + jnp.log(inner)
    out_ref[:, :] = jnp.sum(lse - new_zy, keepdims=True) * (1.0 / B)


_sc_call = functools.partial(
    pl.kernel,
    out_type=(
        jax.ShapeDtypeStruct((NW * B,), jnp.float32),
        jax.ShapeDtypeStruct((NW * B,), jnp.float32),
        jax.ShapeDtypeStruct((NW * B,), jnp.float32),
    ),
    mesh=plsc.VectorSubcoreMesh(core_axis_name="c", subcore_axis_name="s"),
    scratch_types=[
        pltpu.VMEM((CHC, B), jnp.float32),
        pltpu.VMEM((CHC, B), jnp.float32),
        pltpu.VMEM((B,), jnp.int32),
        pltpu.VMEM((B,), jnp.float32),
        pltpu.VMEM((B,), jnp.float32),
        pltpu.VMEM((B,), jnp.float32),
        pltpu.SemaphoreType.DMA,
        pltpu.SemaphoreType.DMA,
    ],
)(_sc_kernel)


def kernel(fc7, weight, nembedding, target):
    ft = fc7.T
    tgt = target.astype(jnp.int32)

    m_sc, s_sc, zy_sc = _sc_call(ft, tgt)

    m_tc, s_tc, zy_tc = pl.pallas_call(
        _tc_kernel,
        grid=(NBLK,),
        in_specs=[
            pl.BlockSpec((1, B), lambda i: (0, 0)),
            pl.BlockSpec((CBLK, B), lambda i: (i + BLK0, 0)),
        ],
        out_specs=[
            pl.BlockSpec((1, B), lambda i: (0, 0)),
            pl.BlockSpec((1, B), lambda i: (0, 0)),
            pl.BlockSpec((1, B), lambda i: (0, 0)),
        ],
        out_shape=[
            jax.ShapeDtypeStruct((1, B), jnp.float32),
            jax.ShapeDtypeStruct((1, B), jnp.float32),
            jax.ShapeDtypeStruct((1, B), jnp.float32),
        ],
    )(tgt.reshape(1, B), ft)

    out = pl.pallas_call(
        _combine_kernel,
        out_shape=jax.ShapeDtypeStruct((1, 1), jnp.float32),
    )(
        m_tc,
        s_tc,
        zy_tc,
        m_sc.reshape(NW, B),
        s_sc.reshape(NW, B),
        zy_sc.reshape(NW, B),
    )
    return out[0, 0]


# speedup vs baseline: 1.0494x; 1.0494x over previous
import functools
import math

import jax
import jax.numpy as jnp
from jax import lax
from jax.experimental import pallas as pl
from jax.experimental.pallas import tpu as pltpu
from jax.experimental.pallas import tpu_sc as plsc

B = 1024
V = 100000
SCALE = 64.0
COS_M2 = math.cos(0.5)
SIN_M2 = math.sin(0.5)
NEG = -1e30

C_SC = 28672

CBLK = 2048
BLK0 = C_SC // CBLK
NBLK = (V - C_SC + CBLK - 1) // CBLK

NC = 2
NS = 16
NW = NC * NS
LANES = 16
NSTRIP = B // LANES
CPW = C_SC // NW
CHC = 32
NCHK = CPW // CHC


def _tc_kernel(tgt_ref, x_ref, m_ref, s_ref, zy_ref):
    pid = pl.program_id(0)

    @pl.when(pid == 0)
    def _init():
        m_ref[:, :] = jnp.full((1, B), NEG, jnp.float32)
        s_ref[:, :] = jnp.zeros((1, B), jnp.float32)
        zy_ref[:, :] = jnp.zeros((1, B), jnp.float32)

    x = x_ref[:, :]
    rows = lax.broadcasted_iota(jnp.int32, (CBLK, B), 0)

    def _step(xm):
        m_old = m_ref[:, :]
        bm = jnp.max(xm, axis=0, keepdims=True)
        m_new = jnp.maximum(m_old, bm)
        e = jnp.exp(xm - m_new)
        s_ref[:, :] = s_ref[:, :] * jnp.exp(m_old - m_new) + jnp.sum(
            e, axis=0, keepdims=True
        )
        m_ref[:, :] = m_new

        rel = tgt_ref[:, :] - (C_SC + pid * CBLK)
        zy_ref[:, :] = zy_ref[:, :] + jnp.sum(
            jnp.where(rows == rel, x, 0.0), axis=0, keepdims=True
        )

    @pl.when(pid < NBLK - 1)
    def _full():
        _step(x)

    @pl.when(pid == NBLK - 1)
    def _tail():
        _step(jnp.where(rows < V - C_SC - pid * CBLK, x, NEG))


def _sc_kernel(ft_ref, tgt_ref, m_out, s_out, zy_out,
               chunk_a, chunk_b, tgt_v, m_acc, s_acc, zy_acc, sem_a, sem_b):
    wid = lax.axis_index("s") * NC + lax.axis_index("c")
    c0 = pl.multiple_of(wid * CPW, CPW)

    pltpu.sync_copy(tgt_ref, tgt_v)

    zero = jnp.zeros((LANES,), jnp.float32)
    neg = jnp.full((LANES,), NEG, jnp.float32)
    for i in range(NSTRIP):
        m_acc[pl.ds(i * LANES, LANES)] = neg
        s_acc[pl.ds(i * LANES, LANES)] = zero
        zy_acc[pl.ds(i * LANES, LANES)] = zero

    def _start(ch, buf, sem):
        colr = pl.multiple_of(c0 + ch * CHC, 8)
        pltpu.async_copy(ft_ref.at[pl.ds(colr, CHC), pl.ds(0, B)], buf, sem)

    def _wait(buf, sem):
        pltpu.make_async_copy(
            ft_ref.at[pl.ds(0, CHC), pl.ds(0, B)], buf, sem
        ).wait()

    def _process(chunk_v, ch):
        def strip_body(st, _):
            off = st * LANES
            m_old = m_acc[pl.ds(off, LANES)]
            s_old = s_acc[pl.ds(off, LANES)]
            z_old = zy_acc[pl.ds(off, LANES)]
            t16 = tgt_v[pl.ds(off, LANES)]
            rel16 = t16 - (c0 + ch * CHC)

            a0 = a1 = a2 = a3 = neg
            z = z_old
            for c in range(CHC):
                v = chunk_v[c, pl.ds(off, LANES)]
                z = z + jnp.where(rel16 == c, v, zero)
                if c % 4 == 0:
                    a0 = jnp.maximum(a0, v)
                elif c % 4 == 1:
                    a1 = jnp.maximum(a1, v)
                elif c % 4 == 2:
                    a2 = jnp.maximum(a2, v)
                else:
                    a3 = jnp.maximum(a3, v)
            bm = jnp.maximum(jnp.maximum(a0, a1), jnp.maximum(a2, a3))
            m_new = jnp.maximum(m_old, bm)
            s_scaled = s_old * jnp.exp(m_old - m_new)

            s0 = s1 = s2 = s3 = zero
            for c in range(CHC):
                e = jnp.exp(chunk_v[c, pl.ds(off, LANES)] - m_new)
                if c % 4 == 0:
                    s0 = s0 + e
                elif c % 4 == 1:
                    s1 = s1 + e
                elif c % 4 == 2:
                    s2 = s2 + e
                else:
                    s3 = s3 + e
            s_new = s_scaled + ((s0 + s1) + (s2 + s3))

            m_acc[pl.ds(off, LANES)] = m_new
            s_acc[pl.ds(off, LANES)] = s_new
            zy_acc[pl.ds(off, LANES)] = z
            return 0

        lax.fori_loop(0, NSTRIP, strip_body, 0)

    _start(0, chunk_a, sem_a)

    def pair_body(i, _):
        _start(2 * i + 1, chunk_b, sem_b)
        _wait(chunk_a, sem_a)
        _process(chunk_a, 2 * i)

        @pl.when(i < NCHK // 2 - 1)
        def _():
            _start(2 * i + 2, chunk_a, sem_a)

        _wait(chunk_b, sem_b)
        _process(chunk_b, 2 * i + 1)
        return 0

    lax.fori_loop(0, NCHK // 2, pair_body, 0)

    pltpu.sync_copy(m_acc, m_out.at[pl.ds(wid * B, B)])
    pltpu.sync_copy(s_acc, s_out.at[pl.ds(wid * B, B)])
    pltpu.sync_copy(zy_acc, zy_out.at[pl.ds(wid * B, B)])


def _combine_kernel(mtc_ref, stc_ref, zytc_ref, msc_ref, ssc_ref, zysc_ref,
                    out_ref):
    m16 = msc_ref[:, :]
    s16 = ssc_ref[:, :]
    m_sc = jnp.max(m16, axis=0, keepdims=True)
    s_sc = jnp.sum(s16 * jnp.exp(m16 - m_sc), axis=0, keepdims=True)
    zy = zytc_ref[:, :] + jnp.sum(zysc_ref[:, :], axis=0, keepdims=True)
    m_tc = mtc_ref[:, :]
    s_tc = stc_ref[:, :]
    m = jnp.maximum(m_tc, m_sc)
    s = s_tc * jnp.exp(m_tc - m) + s_sc * jnp.exp(m_sc - m)
    c = zy * (1.0 / SCALE)
    new_zy = SCALE * (c * COS_M2 - jnp.sqrt(1.0 - c * c) * SIN_M2)
    m2 = jnp.maximum(m, new_zy)
    inner = s * jnp.exp(m - m2) - jnp.exp(zy - m2) + jnp.exp(new_zy - m2)
    lse = m2 + jnp.log(inner)
    out_ref[:, :] = jnp.sum(lse - new_zy, keepdims=True) * (1.0 / B)


_sc_call = functools.partial(
    pl.kernel,
    out_type=(
        jax.ShapeDtypeStruct((NW * B,), jnp.float32),
        jax.ShapeDtypeStruct((NW * B,), jnp.float32),
        jax.ShapeDtypeStruct((NW * B,), jnp.float32),
    ),
    mesh=plsc.VectorSubcoreMesh(core_axis_name="c", subcore_axis_name="s"),
    scratch_types=[
        pltpu.VMEM((CHC, B), jnp.float32),
        pltpu.VMEM((CHC, B), jnp.float32),
        pltpu.VMEM((B,), jnp.int32),
        pltpu.VMEM((B,), jnp.float32),
        pltpu.VMEM((B,), jnp.float32),
        pltpu.VMEM((B,), jnp.float32),
        pltpu.SemaphoreType.DMA,
        pltpu.SemaphoreType.DMA,
    ],
)(_sc_kernel)


def kernel(fc7, weight, nembedding, target):
    ft = fc7.T
    tgt = target.astype(jnp.int32)

    m_sc, s_sc, zy_sc = _sc_call(ft, tgt)

    m_tc, s_tc, zy_tc = pl.pallas_call(
        _tc_kernel,
        grid=(NBLK,),
        in_specs=[
            pl.BlockSpec((1, B), lambda i: (0, 0)),
            pl.BlockSpec((CBLK, B), lambda i: (i + BLK0, 0)),
        ],
        out_specs=[
            pl.BlockSpec((1, B), lambda i: (0, 0)),
            pl.BlockSpec((1, B), lambda i: (0, 0)),
            pl.BlockSpec((1, B), lambda i: (0, 0)),
        ],
        out_shape=[
            jax.ShapeDtypeStruct((1, B), jnp.float32),
            jax.ShapeDtypeStruct((1, B), jnp.float32),
            jax.ShapeDtypeStruct((1, B), jnp.float32),
        ],
    )(tgt.reshape(1, B), ft)

    out = pl.pallas_call(
        _combine_kernel,
        out_shape=jax.ShapeDtypeStruct((1, 1), jnp.float32),
    )(
        m_tc,
        s_tc,
        zy_tc,
        m_sc.reshape(NW, B),
        s_sc.reshape(NW, B),
        zy_sc.reshape(NW, B),
    )
    return out[0, 0]
